# baseline (speedup 1.0000x reference)
import functools

import jax
import jax.numpy as jnp
from jax import lax
from jax.experimental import pallas as pl
from jax.experimental.pallas import tpu as pltpu
from jax.experimental.pallas import tpu_sc as plsc

B, L, V, D, DO = 16384, 200, 165, 4, 10
NC, NS = 2, 16
NW = NC * NS
RPW = B // NW
BLK = 16
W4 = L // 4
W8 = 56
WP = W8 + 1
UNROLL = 5


def _make_kernel():
    mesh = plsc.VectorSubcoreMesh(
        core_axis_name="c", subcore_axis_name="s", num_cores=NC,
        num_subcores=NS)

    @functools.partial(
        pl.kernel,
        out_type=jax.ShapeDtypeStruct((B, DO), jnp.float32),
        mesh=mesh,
        compiler_params=pltpu.CompilerParams(
            use_tc_tiling_on_sc=False, needs_layout_passes=False),
        scratch_types=[
            pltpu.VMEM((RPW, WP), jnp.int32),
            pltpu.VMEM((V * D * 16,), jnp.float32),
            pltpu.VMEM((D, 16), jnp.float32),
            pltpu.VMEM((16,), jnp.float32),
            pltpu.VMEM((RPW, DO), jnp.float32),
        ],
    )
    def showdown_kernel(x_hbm, tbl_hbm, w_hbm, b_hbm, out_hbm,
                        x_v, tbl_v, w_v, b_v, out_v):
        wid = lax.axis_index("s") * NC + lax.axis_index("c")
        base = wid * RPW
        pltpu.sync_copy(x_hbm.at[pl.ds(base, RPW)], x_v.at[:, pl.ds(0, W8)])
        pltpu.sync_copy(tbl_hbm, tbl_v)
        pltpu.sync_copy(w_hbm, w_v)
        pltpu.sync_copy(b_hbm, b_v)

        lane = lax.iota(jnp.int32, 16)
        wrows = [w_v[d] for d in range(D)]
        brow = b_v[...]
        wj = [[wrows[d][j] for j in range(DO)] for d in range(D)]
        bj = [brow[j] for j in range(DO)]
        jvecs = [jnp.full((16,), j, jnp.int32) for j in range(DO)]
        doffs = [lane + d * 16 for d in range(D)]
        m255 = jnp.full((16,), 255, jnp.int32)

        def block_body(blk, _):
            rows = lane + blk * BLK
            zero = jnp.zeros((16,), jnp.float32)

            def w_body(i, accs):
                accs = list(accs)
                for k in range(UNROLL):
                    wsplat = jnp.full((16,), i * UNROLL + k, jnp.int32)
                    word = plsc.load_gather(x_v, [rows, wsplat])
                    for s in range(4):
                        idx = (word >> (8 * s)) & m255 if s else word & m255
                        idx64 = idx * (D * 16)
                        for d in range(D):
                            g = plsc.load_gather(tbl_v, [idx64 + doffs[d]])
                            accs[d] = accs[d] + g
                return tuple(accs)

            accs = lax.fori_loop(0, W4 // UNROLL, w_body,
                                 (zero, zero, zero, zero))
            for j in range(DO):
                o = jnp.full((16,), bj[j], jnp.float32)
                for d in range(D):
                    o = o + accs[d] * wj[d][j]
                plsc.store_scatter(out_v, [rows, jvecs[j]], o)
            return 0

        lax.fori_loop(0, RPW // BLK, block_body, 0)
        pltpu.sync_copy(out_v, out_hbm.at[pl.ds(base, RPW)])

    return showdown_kernel


_kernel = _make_kernel()


def kernel(x, embed_table, W, b):
    xi = x.astype(jnp.int32)
    q = W4 * 2
    xp = (xi[:, :W4] | (xi[:, W4:q] << 8) | (xi[:, q:q + W4] << 16)
          | (xi[:, q + W4:] << 24))
    xp = jnp.pad(xp, ((0, 0), (0, W8 - W4)))
    tbl_rep = jnp.broadcast_to(
        embed_table.reshape(V * D, 1), (V * D, 16)).reshape(-1)
    w_pad = jnp.zeros((D, 16), jnp.float32).at[:, :DO].set(W)
    b_pad = jnp.zeros((16,), jnp.float32).at[:DO].set(b)
    return _kernel(xp, tbl_rep, w_pad, b_pad)


# speedup vs baseline: 1.7917x; 1.7917x over previous
import functools

import jax
import jax.numpy as jnp
from jax import lax
from jax.experimental import pallas as pl
from jax.experimental.pallas import tpu as pltpu
from jax.experimental.pallas import tpu_sc as plsc

B, L, V, D, DO = 16384, 200, 165, 4, 10
NC, NS = 2, 16
NW = NC * NS
Q = B // 4
GPW = Q // NW
RPW = 4 * GPW
BLK = 16
UNROLL = 4


def _make_kernel():
    mesh = plsc.VectorSubcoreMesh(
        core_axis_name="c", subcore_axis_name="s", num_cores=NC,
        num_subcores=NS)

    @functools.partial(
        pl.kernel,
        out_type=jax.ShapeDtypeStruct((B, DO), jnp.float32),
        mesh=mesh,
        compiler_params=pltpu.CompilerParams(
            use_tc_tiling_on_sc=False, needs_layout_passes=False),
        scratch_types=[
            pltpu.VMEM((GPW * L,), jnp.int32),
            pltpu.VMEM((V * D * 16,), jnp.float32),
            pltpu.VMEM((D, 16), jnp.float32),
            pltpu.VMEM((16,), jnp.float32),
            pltpu.VMEM((RPW, DO), jnp.float32),
        ],
    )
    def showdown_kernel(x_hbm, tbl_hbm, w_hbm, b_hbm, out_hbm,
                        x_v, tbl_v, w_v, b_v, out_v):
        wid = lax.axis_index("s") * NC + lax.axis_index("c")
        pltpu.sync_copy(x_hbm.at[pl.ds(wid * (GPW * L), GPW * L)], x_v)
        pltpu.sync_copy(tbl_hbm, tbl_v)
        pltpu.sync_copy(w_hbm, w_v)
        pltpu.sync_copy(b_hbm, b_v)

        lane = lax.iota(jnp.int32, 16)
        wrows = [w_v[d] for d in range(D)]
        brow = b_v[...]
        wj = [[wrows[d][j] for j in range(DO)] for d in range(D)]
        bj = [brow[j] for j in range(DO)]
        jvecs = [jnp.full((16,), j, jnp.int32) for j in range(DO)]
        doffs = [lane + d * 16 for d in range(D)]
        m255 = jnp.full((16,), 255, jnp.int32)
        wrap = jnp.full((16,), L, jnp.int32)

        zero = jnp.zeros((16,), jnp.float32)
        zeros16 = tuple(zero for _ in range(16))

        def block_body(blk, _):
            groups = lane + blk * BLK
            addr0 = groups * L + lane
            end = groups * L + L

            def l_body(i, carry):
                addr, cnt = carry[0], carry[1]
                accs = list(carry[2:])
                for _ in range(UNROLL):
                    word = plsc.load_gather(x_v, [addr])
                    for s in range(4):
                        idx = (word >> (8 * s)) & m255 if s else word & m255
                        idx64 = idx * (D * 16)
                        for d in range(D):
                            g = plsc.load_gather(tbl_v, [idx64 + doffs[d]])
                            a = 4 * s + d
                            accs[a] = accs[a] + g
                    cnt = cnt + 1
                    addr = addr + 1
                    addr = jnp.where(cnt == wrap, addr - L, addr)
                return (addr, cnt) + tuple(accs)

            carry = lax.fori_loop(
                0, L // UNROLL, l_body,
                (addr0, lane, *zeros16))
            accs = carry[2:]
            for s in range(4):
                rows_s = groups + s * GPW
                for j in range(DO):
                    o = jnp.full((16,), bj[j], jnp.float32)
                    for d in range(D):
                        o = o + accs[4 * s + d] * wj[d][j]
                    plsc.store_scatter(out_v, [rows_s, jvecs[j]], o)
            return 0

        lax.fori_loop(0, GPW // BLK, block_body, 0)
        for s in range(4):
            pltpu.sync_copy(
                out_v.at[pl.ds(s * GPW, GPW)],
                out_hbm.at[pl.ds(wid * GPW + s * Q, GPW)])

    return showdown_kernel


_kernel = _make_kernel()


def kernel(x, embed_table, W, b):
    xi = x.astype(jnp.int32)
    xp = (xi[:Q] | (xi[Q:2 * Q] << 8) | (xi[2 * Q:3 * Q] << 16)
          | (xi[3 * Q:] << 24))
    xp1 = xp.reshape(-1)
    tbl_rep = jnp.broadcast_to(
        embed_table.reshape(V * D, 1), (V * D, 16)).reshape(-1)
    w_pad = jnp.zeros((D, 16), jnp.float32).at[:, :DO].set(W)
    b_pad = jnp.zeros((16,), jnp.float32).at[:DO].set(b)
    return _kernel(xp1, tbl_rep, w_pad, b_pad)


# speedup vs baseline: 1.8228x; 1.0173x over previous
import functools

import jax
import jax.numpy as jnp
from jax import lax
from jax.experimental import pallas as pl
from jax.experimental.pallas import tpu as pltpu
from jax.experimental.pallas import tpu_sc as plsc

B, L, V, D, DO = 16384, 200, 165, 4, 10
NC, NS = 2, 16
NW = NC * NS
Q = B // 4
GPW = Q // NW
RPW = 4 * GPW
BLK = 16
XS = 256
UNROLL = 4
TBL = V * D * 16


def _make_kernel():
    mesh = plsc.VectorSubcoreMesh(
        core_axis_name="c", subcore_axis_name="s", num_cores=NC,
        num_subcores=NS)

    @functools.partial(
        pl.kernel,
        out_type=jax.ShapeDtypeStruct((B * DO,), jnp.float32),
        mesh=mesh,
        compiler_params=pltpu.CompilerParams(
            use_tc_tiling_on_sc=False, needs_layout_passes=False),
        scratch_types=[
            pltpu.VMEM((GPW * XS,), jnp.int32),
            pltpu.VMEM((TBL + 64,), jnp.float32),
            pltpu.VMEM((D, 16), jnp.float32),
            pltpu.VMEM((16,), jnp.float32),
            pltpu.VMEM((RPW * DO,), jnp.float32),
        ],
    )
    def showdown_kernel(x_hbm, tbl_hbm, w_hbm, b_hbm, out_hbm,
                        x_v, tbl_v, w_v, b_v, out_v):
        wid = lax.axis_index("s") * NC + lax.axis_index("c")
        pltpu.sync_copy(x_hbm.at[pl.ds(wid * (GPW * XS), GPW * XS)], x_v)
        pltpu.sync_copy(tbl_hbm, tbl_v.at[pl.ds(0, TBL)])
        pltpu.sync_copy(w_hbm, w_v)
        pltpu.sync_copy(b_hbm, b_v)

        lane = lax.iota(jnp.int32, 16)
        wrows = [w_v[d] for d in range(D)]
        brow = b_v[...]
        wj = [[wrows[d][j] for j in range(DO)] for d in range(D)]
        bj = [brow[j] for j in range(DO)]
        tbl_d = [tbl_v.at[pl.ds(d * 16, TBL)] for d in range(D)]
        mhi = jnp.full((16,), 255 << 6, jnp.int32)
        wrap = jnp.full((16,), L, jnp.int32)

        zero = jnp.zeros((16,), jnp.float32)
        zeros16 = tuple(zero for _ in range(16))

        def block_body(blk, _):
            groups = lane + blk * BLK
            addr0 = groups * XS + lane

            def l_body(i, carry):
                addr, cnt = carry[0], carry[1]
                accs = list(carry[2:])
                for _ in range(UNROLL):
                    word = plsc.load_gather(x_v, [addr])
                    for s in range(4):
                        sh = 8 * s - 6
                        hi = ((word << 6) if sh < 0 else (word >> sh)) & mhi
                        gidx = hi | lane
                        for d in range(D):
                            g = plsc.load_gather(tbl_d[d], [gidx])
                            a = 4 * s + d
                            accs[a] = accs[a] + g
                    cnt = cnt + 1
                    addr = addr + 1
                    addr = jnp.where(cnt == wrap, addr - L, addr)
                return (addr, cnt) + tuple(accs)

            carry = lax.fori_loop(
                0, L // UNROLL, l_body,
                (addr0, lane, *zeros16))
            accs = carry[2:]
            for s in range(4):
                rbase = (groups + s * GPW) * DO
                for j in range(DO):
                    o = jnp.full((16,), bj[j], jnp.float32)
                    for d in range(D):
                        o = o + accs[4 * s + d] * wj[d][j]
                    plsc.store_scatter(out_v, [rbase + j], o)
            return 0

        lax.fori_loop(0, GPW // BLK, block_body, 0)
        for s in range(4):
            pltpu.sync_copy(
                out_v.at[pl.ds(s * GPW * DO, GPW * DO)],
                out_hbm.at[pl.ds((wid * GPW + s * Q) * DO, GPW * DO)])

    return showdown_kernel


_kernel = _make_kernel()


def kernel(x, embed_table, W, b):
    xi = x.astype(jnp.int32)
    xp = (xi[:Q] | (xi[Q:2 * Q] << 8) | (xi[2 * Q:3 * Q] << 16)
          | (xi[3 * Q:] << 24))
    xp1 = jnp.pad(xp, ((0, 0), (0, XS - L))).reshape(-1)
    tbl_rep = jnp.broadcast_to(
        embed_table.reshape(V * D, 1), (V * D, 16)).reshape(-1)
    w_pad = jnp.concatenate([W, jnp.zeros((D, 16 - DO), jnp.float32)], 1)
    b_pad = jnp.concatenate([b, jnp.zeros((16 - DO,), jnp.float32)])
    out = _kernel(xp1, tbl_rep, w_pad, b_pad)
    return out.reshape(B, DO)


# speedup vs baseline: 1.8409x; 1.0100x over previous
import functools

import jax
import jax.numpy as jnp
from jax import lax
from jax.experimental import pallas as pl
from jax.experimental.pallas import tpu as pltpu
from jax.experimental.pallas import tpu_sc as plsc

B, L, V, D, DO = 16384, 200, 165, 4, 10
NC, NS = 2, 16
NW = NC * NS
Q = B // 4
GPW = Q // NW
RPW = 4 * GPW
BLK = 16
XS = 256
UNROLL = 4
TBL = V * D * 16


def _make_kernel():
    mesh = plsc.VectorSubcoreMesh(
        core_axis_name="c", subcore_axis_name="s", num_cores=NC,
        num_subcores=NS)

    @functools.partial(
        pl.kernel,
        out_type=jax.ShapeDtypeStruct((B, DO), jnp.float32),
        mesh=mesh,
        compiler_params=pltpu.CompilerParams(
            use_tc_tiling_on_sc=False, needs_layout_passes=False),
        scratch_types=[
            pltpu.VMEM((GPW * XS,), jnp.int32),
            pltpu.VMEM((TBL + 64,), jnp.float32),
            pltpu.VMEM((D, 16), jnp.float32),
            pltpu.VMEM((16,), jnp.float32),
            pltpu.VMEM((RPW, DO), jnp.float32),
            pltpu.SemaphoreType.DMA,
            pltpu.SemaphoreType.DMA,
        ],
    )
    def showdown_kernel(x_hbm, tbl_hbm, w_hbm, b_hbm, out_hbm,
                        x_v, tbl_v, w_v, b_v, out_v, sem0, sem1):
        wid = lax.axis_index("s") * NC + lax.axis_index("c")
        half = GPW * XS // 2
        c0 = pltpu.async_copy(
            x_hbm.at[pl.ds(wid * (GPW * XS), half)],
            x_v.at[pl.ds(0, half)], sem0)
        c1 = pltpu.async_copy(
            x_hbm.at[pl.ds(wid * (GPW * XS) + half, half)],
            x_v.at[pl.ds(half, half)], sem1)
        pltpu.sync_copy(tbl_hbm, tbl_v.at[pl.ds(0, TBL)])
        pltpu.sync_copy(w_hbm, w_v)
        pltpu.sync_copy(b_hbm, b_v)
        c0.wait()

        lane = lax.iota(jnp.int32, 16)
        wrows = [w_v[d] for d in range(D)]
        brow = b_v[...]
        wj = [[wrows[d][j] for j in range(DO)] for d in range(D)]
        bj = [brow[j] for j in range(DO)]
        tbl_d = [tbl_v.at[pl.ds(d * 16, TBL)] for d in range(D)]
        jvecs = [jnp.full((16,), j, jnp.int32) for j in range(DO)]
        mhi = jnp.full((16,), 255 << 6, jnp.int32)
        wrap = jnp.full((16,), L, jnp.int32)

        zero = jnp.zeros((16,), jnp.float32)
        zeros16 = tuple(zero for _ in range(16))

        def block_body(blk, _):
            groups = lane + blk * BLK
            addr0 = groups * XS + lane

            def l_body(i, carry):
                addr, cnt = carry[0], carry[1]
                accs = list(carry[2:])
                for _ in range(UNROLL):
                    word = plsc.load_gather(x_v, [addr])
                    for s in range(4):
                        sh = 8 * s - 6
                        hi = ((word << 6) if sh < 0 else (word >> sh)) & mhi
                        gidx = hi | lane
                        for d in range(D):
                            g = plsc.load_gather(tbl_d[d], [gidx])
                            a = 4 * s + d
                            accs[a] = accs[a] + g
                    cnt = cnt + 1
                    addr = addr + 1
                    addr = jnp.where(cnt == wrap, addr - L, addr)
                return (addr, cnt) + tuple(accs)

            carry = lax.fori_loop(
                0, L // UNROLL, l_body,
                (addr0, lane, *zeros16))
            accs = carry[2:]
            for s in range(4):
                rows_s = groups + s * GPW
                for j in range(DO):
                    o = jnp.full((16,), bj[j], jnp.float32)
                    for d in range(D):
                        o = o + accs[4 * s + d] * wj[d][j]
                    plsc.store_scatter(out_v, [rows_s, jvecs[j]], o)
            return 0

        lax.fori_loop(0, GPW // BLK // 2, block_body, 0)
        c1.wait()
        lax.fori_loop(GPW // BLK // 2, GPW // BLK, block_body, 0)
        for s in range(4):
            pltpu.sync_copy(
                out_v.at[pl.ds(s * GPW, GPW)],
                out_hbm.at[pl.ds(wid * GPW + s * Q, GPW)])

    return showdown_kernel


_kernel = _make_kernel()


def kernel(x, embed_table, W, b):
    xi = x.astype(jnp.int32)
    pad = ((0, 0), (0, XS - L))
    xp = (jnp.pad(xi[:Q], pad) | (jnp.pad(xi[Q:2 * Q], pad) << 8)
          | (jnp.pad(xi[2 * Q:3 * Q], pad) << 16)
          | (jnp.pad(xi[3 * Q:], pad) << 24))
    xp1 = xp.reshape(-1)
    tbl_rep = jnp.broadcast_to(
        embed_table.reshape(V * D, 1), (V * D, 16)).reshape(-1)
    w_pad = jnp.concatenate([W, jnp.zeros((D, 16 - DO), jnp.float32)], 1)
    b_pad = jnp.concatenate([b, jnp.zeros((16 - DO,), jnp.float32)])
    out = _kernel(xp1, tbl_rep, w_pad, b_pad)
    return out.reshape(B, DO)
